# baseline (speedup 1.0000x reference)
import functools

import jax
import jax.numpy as jnp
from jax import lax
from jax.experimental import pallas as pl
from jax.experimental.pallas import tpu as pltpu
from jax.experimental.pallas import tpu_sc as plsc

N_NODES = 10000
N_EDGES = 160000
C = 128
K = 16
NUM_BESSEL = 8
R_MAX = 5.0
P_CUT = 5.0
AVG_N = 16.0

NC = 2
NS = 16
LANES = 16

N_PAD = 10240
E_PAD = 163840
CH = 128
NCHUNK = N_PAD // CH
EPT = E_PAD // NS
EPW = E_PAD // (NC * NS)
GCH = 128
BS = 16
RECV_SENTINEL = N_PAD - 8


def _f32(x):
    return jnp.asarray(x, jnp.float32)


def _pos_gather(pos_pad, send_p, recv_p):
    mesh = plsc.VectorSubcoreMesh(core_axis_name="c", subcore_axis_name="s")

    @functools.partial(
        pl.kernel,
        mesh=mesh,
        compiler_params=pltpu.CompilerParams(use_tc_tiling_on_sc=False, needs_layout_passes=False),
        out_type=(
            jax.ShapeDtypeStruct((E_PAD, 16), jnp.float32),
            jax.ShapeDtypeStruct((E_PAD, 16), jnp.float32),
        ),
        scratch_types=[
            pltpu.VMEM((GCH,), jnp.int32),
            pltpu.VMEM((GCH, 16), jnp.float32),
            pltpu.SemaphoreType.DMA,
        ],
    )
    def run(pos_hbm, send_hbm, recv_hbm, out_s, out_r, idx_v, rows_v, sem):
        wid = lax.axis_index("s") * NC + lax.axis_index("c")
        base = wid * EPW

        def body(g, _):
            off = base + g * GCH
            pltpu.sync_copy(send_hbm.at[pl.ds(off, GCH)], idx_v)
            pltpu.async_copy(pos_hbm.at[idx_v], rows_v, sem).wait()
            pltpu.sync_copy(rows_v, out_s.at[pl.ds(off, GCH)])
            pltpu.sync_copy(recv_hbm.at[pl.ds(off, GCH)], idx_v)
            pltpu.async_copy(pos_hbm.at[idx_v], rows_v, sem).wait()
            pltpu.sync_copy(rows_v, out_r.at[pl.ds(off, GCH)])
            return _

        lax.fori_loop(0, EPW // GCH, body, 0)

    return run(pos_pad, send_p, recv_p)


def _silu(x):
    return x / (1.0 + jnp.exp(-x))


def _edge_w_kernel(ps_ref, pr_ref, m1_ref, m2_ref, w1_ref, w2_ref):
    ps = ps_ref[...]
    pr = pr_ref[...]
    vec = pr - ps
    x = vec[:, 0:1]
    y = vec[:, 1:2]
    z = vec[:, 2:3]
    r = jnp.sqrt(x * x + y * y + z * z) + 1e-9
    xu = x / r
    yu = y / r
    zu = z / r

    s3 = 3.0 ** 0.5
    s15 = 15.0 ** 0.5
    one = jnp.ones_like(xu)
    sh_cols = [
        one,
        s3 * xu, s3 * yu, s3 * zu,
        s15 * xu * yu, s15 * yu * zu,
        (5.0 ** 0.5 / 2.0) * (3.0 * zu * zu - 1.0),
        s15 * xu * zu, (s15 / 2.0) * (xu * xu - yu * yu),
        (35.0 / 8.0) ** 0.5 * yu * (3.0 * xu * xu - yu * yu),
        (105.0 ** 0.5) * xu * yu * zu,
        (21.0 / 8.0) ** 0.5 * yu * (5.0 * zu * zu - 1.0),
        (7.0 ** 0.5 / 2.0) * (5.0 * zu * zu * zu - 3.0 * zu),
        (21.0 / 8.0) ** 0.5 * xu * (5.0 * zu * zu - 1.0),
        (105.0 ** 0.5 / 2.0) * zu * (xu * xu - yu * yu),
        (35.0 / 8.0) ** 0.5 * xu * (xu * xu - 3.0 * yu * yu),
    ]
    sh = jnp.concatenate(sh_cols, axis=1)

    scale = (2.0 / R_MAX) ** 0.5
    bess_cols = []
    for n in range(1, NUM_BESSEL + 1):
        bess_cols.append(jnp.sin(r * (n * jnp.pi / R_MAX)) * (scale / r))
    bess = jnp.concatenate(bess_cols, axis=1)
    u = r / R_MAX
    p = P_CUT
    u2 = u * u
    u4 = u2 * u2
    u5 = u4 * u
    u6 = u4 * u2
    u7 = u6 * u
    env = (1.0
           - (p + 1.0) * (p + 2.0) / 2.0 * u5
           + p * (p + 2.0) * u6
           - p * (p + 1.0) / 2.0 * u7)
    env = jnp.where(u < 1.0, env, 0.0)
    ef = bess * env

    m1 = m1_ref[...]
    m2 = m2_ref[...]
    for idx, (m, w_ref) in enumerate(((m1, w1_ref), (m2, w2_ref))):
        a1 = m[0:8, 0:64]
        a2 = m[8:72, 0:64]
        a3 = m[72:136, 0:64]
        a4 = m[136:200, 0:16]
        h = _silu(jnp.dot(ef, a1, preferred_element_type=jnp.float32))
        h = _silu(jnp.dot(h, a2, preferred_element_type=jnp.float32))
        h = _silu(jnp.dot(h, a3, preferred_element_type=jnp.float32))
        rr = jnp.dot(h, a4, preferred_element_type=jnp.float32)
        w_ref[...] = sh * rr


def _edge_w(pos_s, pos_r, mlp1, mlp2):
    be = 1024
    grid = (E_PAD // be,)
    return pl.pallas_call(
        _edge_w_kernel,
        grid=grid,
        in_specs=[
            pl.BlockSpec((be, 16), lambda i: (i, 0)),
            pl.BlockSpec((be, 16), lambda i: (i, 0)),
            pl.BlockSpec((200, 64), lambda i: (0, 0)),
            pl.BlockSpec((200, 64), lambda i: (0, 0)),
        ],
        out_specs=[
            pl.BlockSpec((be, K), lambda i: (i, 0)),
            pl.BlockSpec((be, K), lambda i: (i, 0)),
        ],
        out_shape=[
            jax.ShapeDtypeStruct((E_PAD, K), jnp.float32),
            jax.ShapeDtypeStruct((E_PAD, K), jnp.float32),
        ],
    )(pos_s, pos_r, mlp1, mlp2)


def _embed_kernel(sp_ref, w_ref, o_ref):
    sp = sp_ref[...]
    io = lax.broadcasted_iota(jnp.int32, (sp.shape[0], 16), 1).astype(
        jnp.float32)
    oh = (sp == io).astype(jnp.float32)
    o_ref[...] = jnp.dot(oh, w_ref[...], preferred_element_type=jnp.float32)


def _embed(species_f, w_pad):
    bn = 1024
    return pl.pallas_call(
        _embed_kernel,
        grid=(N_PAD // bn,),
        in_specs=[
            pl.BlockSpec((bn, 1), lambda i: (i, 0)),
            pl.BlockSpec((16, C), lambda i: (0, 0)),
        ],
        out_specs=pl.BlockSpec((bn, C), lambda i: (i, 0)),
        out_shape=jax.ShapeDtypeStruct((N_PAD, C), jnp.float32),
    )(species_f, w_pad)


def _msg_agg(feats, send_p, recv_p, w_pad):
    mesh = plsc.VectorSubcoreMesh(core_axis_name="c", subcore_axis_name="s")
    ebuf = EPT + BS
    mbuf = EPT + 3 * BS
    rpt = CH // NS

    @functools.partial(
        pl.kernel,
        mesh=mesh,
        compiler_params=pltpu.CompilerParams(
            use_tc_tiling_on_sc=False, needs_layout_passes=False),
        out_type=jax.ShapeDtypeStruct((N_PAD, K * C), jnp.float32),
        scratch_types=[
            pltpu.VMEM((ebuf,), jnp.int32),
            pltpu.VMEM((ebuf,), jnp.int32),
            pltpu.VMEM((mbuf,), jnp.int32),
            pltpu.VMEM((2, BS, C), jnp.float32),
            pltpu.VMEM((2, BS, K), jnp.float32),
            pltpu.VMEM((2, BS, K * C), jnp.float32),
            pltpu.VMEM((1, K * C), jnp.float32),
            pltpu.VMEM_SHARED((CH, K * C), jnp.float32),
            pltpu.SemaphoreType.DMA,
            pltpu.SemaphoreType.DMA,
            pltpu.SemaphoreType.DMA,
            pltpu.SemaphoreType.DMA,
            pltpu.SemaphoreType.DMA,
            pltpu.SemaphoreType.DMA,
        ],
    )
    def run(feats_hbm, send_hbm, recv_hbm, w_hbm, agg_hbm,
            sendbuf, recvbuf, meid, srcb, wb, msg, zbuf,
            agg_s, gf0, gf1, gw0, gw1, sc0, sc1):
        cc = lax.axis_index("c")
        ss = lax.axis_index("s")
        tilebase = ss * EPT
        gf = (gf0, gf1)
        gw = (gw0, gw1)
        sca = (sc0, sc1)

        def zinit(j, _):
            zbuf[0, pl.ds(j * LANES, LANES)] = jnp.zeros((LANES,), jnp.float32)
            return _
        lax.fori_loop(0, (K * C) // LANES, zinit, 0)

        pltpu.sync_copy(send_hbm.at[pl.ds(tilebase, EPT)],
                        sendbuf.at[pl.ds(0, EPT)])
        pltpu.sync_copy(recv_hbm.at[pl.ds(tilebase, EPT)],
                        recvbuf.at[pl.ds(0, EPT)])
        sendbuf[pl.ds(EPT, LANES)] = jnp.zeros((LANES,), jnp.int32)

        def fire_gather(b, which):
            eidx = meid[pl.ds(b * BS, BS)]
            rel = jnp.minimum(eidx - tilebase, EPT)
            sidx = plsc.load_gather(sendbuf, [rel])
            pltpu.async_copy(feats_hbm.at[sidx], srcb.at[which], gf[which])
            pltpu.async_copy(w_hbm.at[eidx], wb.at[which], gw[which])

        def pass_body(p_idx, _):
            chunk = NC * p_idx + cc
            nbase = chunk * CH
            recvbuf[pl.ds(EPT, LANES)] = jnp.full((LANES,), nbase, jnp.int32)

            def zrow(i, _):
                pltpu.sync_copy(zbuf, agg_s.at[pl.ds(ss * rpt + i, 1)])
                return _
            lax.fori_loop(0, rpt, zrow, 0)
            plsc.subcore_barrier()

            def scan_body(g, cnt):
                rv = recvbuf[pl.ds(g * LANES, LANES)]
                m = (rv >= nbase) & (rv < nbase + CH)
                mi = m.astype(jnp.int32)
                dest = cnt + plsc.cumsum(mi) - 1
                eid = (tilebase + g * LANES) + lax.iota(jnp.int32, LANES)
                plsc.store_scatter(meid, [dest], eid, mask=m)
                return cnt + jnp.sum(mi)

            cnt = lax.fori_loop(0, EPT // LANES, scan_body, 0)

            sent = jnp.full((LANES,), E_PAD, jnp.int32)
            meid[pl.ds(cnt, LANES)] = sent
            meid[pl.ds(cnt + LANES, LANES)] = sent
            meid[pl.ds(cnt + 2 * LANES, LANES)] = sent

            nb = (cnt + BS - 1) // BS
            nb2 = jnp.maximum(nb + (nb % 2), 2)

            fire_gather(0, 0)

            def step(q, cur):
                nxt = 1 - cur
                fire_gather(q + 1, nxt)
                pltpu.make_async_copy(
                    feats_hbm.at[pl.ds(0, BS)], srcb.at[cur], gf[cur]).wait()
                pltpu.make_async_copy(
                    w_hbm.at[pl.ds(0, BS)], wb.at[cur], gw[cur]).wait()
                @pl.when(q >= 2)
                def _():
                    pltpu.make_async_copy(
                        msg.at[cur], agg_s.at[pl.ds(0, BS)], sca[cur]).wait()

                def edge_body(e, _):
                    srow = [srcb[cur, e, pl.ds(j * LANES, LANES)]
                            for j in range(8)]
                    for k in range(K):
                        wspl = plsc.load_gather(
                            wb, [jnp.full((LANES,), cur, jnp.int32),
                                 jnp.full((LANES,), e, jnp.int32),
                                 jnp.full((LANES,), k, jnp.int32)])
                        for j in range(8):
                            msg[cur, e, pl.ds(k * C + j * LANES, LANES)] = (
                                wspl * srow[j])
                    return _

                lax.fori_loop(0, BS, edge_body, 0)

                eidx = meid[pl.ds(q * BS, BS)]
                rel = jnp.minimum(eidx - tilebase, EPT)
                ridx = plsc.load_gather(recvbuf, [rel]) - nbase
                pltpu.async_copy(msg.at[cur], agg_s.at[ridx], sca[cur],
                                 add=True)

            def proc_body(q, carry):
                @pl.when(q % 2 == 0)
                def _():
                    step(q, 0)

                @pl.when(q % 2 == 1)
                def _():
                    step(q, 1)
                return carry

            lax.fori_loop(0, nb2, proc_body, 0)

            pltpu.make_async_copy(
                msg.at[0], agg_s.at[pl.ds(0, BS)], sca[0]).wait()
            pltpu.make_async_copy(
                msg.at[1], agg_s.at[pl.ds(0, BS)], sca[1]).wait()
            pltpu.make_async_copy(
                feats_hbm.at[pl.ds(0, BS)], srcb.at[0], gf[0]).wait()
            pltpu.make_async_copy(
                w_hbm.at[pl.ds(0, BS)], wb.at[0], gw[0]).wait()
            plsc.subcore_barrier()

            pltpu.sync_copy(
                agg_s.at[pl.ds(ss * rpt, rpt)],
                agg_hbm.at[pl.ds(nbase + ss * rpt, rpt)])
            return _

        lax.fori_loop(0, NCHUNK // NC, pass_body, 0)

    return run(feats, send_p, recv_p, w_pad)


def _node_update_kernel(agg_ref, f_ref, pp_ref, l_ref, s_ref, o_ref):
    inv_avg = 1.0 / AVG_N
    a = agg_ref[...] * inv_avg
    pp = pp_ref[...]
    p2 = pp[0:1, :]
    p3 = pp[1:2, :]

    m0 = a[:, 0:C]
    inv0 = m0 * m0
    inv1 = (a[:, C:2 * C] * a[:, C:2 * C]
            + a[:, 2 * C:3 * C] * a[:, 2 * C:3 * C]
            + a[:, 3 * C:4 * C] * a[:, 3 * C:4 * C])
    inv2 = sum(a[:, k * C:(k + 1) * C] * a[:, k * C:(k + 1) * C]
               for k in range(4, 9))
    inv3 = sum(a[:, k * C:(k + 1) * C] * a[:, k * C:(k + 1) * C]
               for k in range(9, 16))
    b = (m0 + inv0 * pp[2:3, :] + inv1 * pp[3:4, :]
         + inv2 * pp[4:5, :] + inv3 * pp[5:6, :])
    b = b + (b * b) * p2 + (b * b * b) * p3
    bl = jnp.dot(b, l_ref[...], preferred_element_type=jnp.float32)
    f = f_ref[...]
    o_ref[...] = _silu(bl) + jnp.dot(f, s_ref[...],
                                     preferred_element_type=jnp.float32)


def _node_update(agg, feats, pp, l_mat, s_mat):
    bn = 256
    return pl.pallas_call(
        _node_update_kernel,
        grid=(N_PAD // bn,),
        in_specs=[
            pl.BlockSpec((bn, K * C), lambda i: (i, 0)),
            pl.BlockSpec((bn, C), lambda i: (i, 0)),
            pl.BlockSpec((8, C), lambda i: (0, 0)),
            pl.BlockSpec((C, C), lambda i: (0, 0)),
            pl.BlockSpec((C, C), lambda i: (0, 0)),
        ],
        out_specs=pl.BlockSpec((bn, C), lambda i: (i, 0)),
        out_shape=jax.ShapeDtypeStruct((N_PAD, C), jnp.float32),
    )(agg, feats, pp, l_mat, s_mat)


def _readout_kernel(h1_ref, h2_ref, sp_ref, ae_ref, r1_ref, r2a_ref, r2b_ref,
                    o_ref):
    sp = sp_ref[...]
    io = lax.broadcasted_iota(jnp.int32, (sp.shape[0], 16), 1).astype(
        jnp.float32)
    oh = (sp == io).astype(jnp.float32)
    e0 = jnp.sum(oh * ae_ref[...], axis=1, keepdims=True)
    e1 = jnp.sum(h1_ref[...] * r1_ref[...], axis=1, keepdims=True)
    t = _silu(jnp.dot(h2_ref[...], r2a_ref[...],
                      preferred_element_type=jnp.float32))
    e2 = jnp.sum(t * r2b_ref[...], axis=1, keepdims=True)
    o_ref[...] = e0 + e1 + e2


def _readout(h1, h2, species_f, ae_row, r1_row, r2a, r2b_row):
    bn = 1024
    return pl.pallas_call(
        _readout_kernel,
        grid=(N_PAD // bn,),
        in_specs=[
            pl.BlockSpec((bn, C), lambda i: (i, 0)),
            pl.BlockSpec((bn, C), lambda i: (i, 0)),
            pl.BlockSpec((bn, 1), lambda i: (i, 0)),
            pl.BlockSpec((1, 16), lambda i: (0, 0)),
            pl.BlockSpec((1, C), lambda i: (0, 0)),
            pl.BlockSpec((C, 16), lambda i: (0, 0)),
            pl.BlockSpec((1, 16), lambda i: (0, 0)),
        ],
        out_specs=pl.BlockSpec((bn, 1), lambda i: (i, 0)),
        out_shape=jax.ShapeDtypeStruct((N_PAD, 1), jnp.float32),
    )(h1, h2, species_f, ae_row, r1_row, r2a, r2b_row)


def _pack_mlp(p):
    m = jnp.zeros((200, 64), jnp.float32)
    m = m.at[0:8, 0:64].set(p['A1'])
    m = m.at[8:72, 0:64].set(p['A2'])
    m = m.at[72:136, 0:64].set(p['A3'])
    m = m.at[136:200, 0:16].set(p['A4'])
    return m


def _pack_pp(p):
    pp = jnp.zeros((8, C), jnp.float32)
    pp = pp.at[0, :].set(p['P2'])
    pp = pp.at[1, :].set(p['P3'])
    pp = pp.at[2:6, :].set(p['C'])
    return pp


def kernel(positions, species, edge_index, params):
    f32 = jnp.float32
    i32 = jnp.int32
    sender = edge_index[0].astype(i32)
    receiver = edge_index[1].astype(i32)
    epad = E_PAD - N_EDGES
    send_p = jnp.concatenate([sender, jnp.zeros((epad,), i32)])
    recv_p = jnp.concatenate(
        [receiver, jnp.full((epad,), RECV_SENTINEL, i32)])

    pos_pad = jnp.pad(positions.astype(f32),
                      ((0, N_PAD - N_NODES), (0, 13)))
    species_f = jnp.pad(species.astype(f32), (0, N_PAD - N_NODES))[:, None]

    p = params
    w_embed_pad = jnp.pad(p['W_embed'].astype(f32), ((0, 6), (0, 0)))
    ae_row = jnp.pad(p['AE'].astype(f32), (0, 6))[None, :]
    r1_row = p['R1'].astype(f32)[:, 0][None, :]
    r2b_row = jnp.pad(p['R2b'].astype(f32)[:, 0], (0, 0))[None, :]

    mlp1 = _pack_mlp(p['int1'])
    mlp2 = _pack_mlp(p['int2'])
    pp1 = _pack_pp(p['int1'])
    pp2 = _pack_pp(p['int2'])

    pos_s, pos_r = _pos_gather(pos_pad, send_p, recv_p)
    w1, w2 = _edge_w(pos_s, pos_r, mlp1, mlp2)
    w1p = jnp.pad(w1, ((0, BS), (0, 0)))
    w2p = jnp.pad(w2, ((0, BS), (0, 0)))

    feats0 = _embed(species_f, w_embed_pad)
    agg1 = _msg_agg(feats0, send_p, recv_p, w1p)
    h1 = _node_update(agg1, feats0, pp1, p['int1']['L'], p['int1']['S'])
    agg2 = _msg_agg(h1, send_p, recv_p, w2p)
    h2 = _node_update(agg2, h1, pp2, p['int2']['L'], p['int2']['S'])

    out = _readout(h1, h2, species_f, ae_row, r1_row, p['R2a'], r2b_row)
    return out[:N_NODES, 0]


# speedup vs baseline: 1.4964x; 1.4964x over previous
import functools

import jax
import jax.numpy as jnp
from jax import lax
from jax.experimental import pallas as pl
from jax.experimental.pallas import tpu as pltpu
from jax.experimental.pallas import tpu_sc as plsc

N_NODES = 10000
N_EDGES = 160000
C = 128
K = 16
NUM_BESSEL = 8
R_MAX = 5.0
P_CUT = 5.0
AVG_N = 16.0

NC = 2
NS = 16
LANES = 16

N_PAD = 10240
E_PAD = 163840
CH = 256
NCHUNK = N_PAD // CH
EPT = E_PAD // NS
EPW = E_PAD // (NC * NS)
GCH = 128
BS = 16
RECV_SENTINEL = N_PAD - 8


def _f32(x):
    return jnp.asarray(x, jnp.float32)


def _pos_gather(pos_pad, send_p, recv_p):
    mesh = plsc.VectorSubcoreMesh(core_axis_name="c", subcore_axis_name="s")

    @functools.partial(
        pl.kernel,
        mesh=mesh,
        compiler_params=pltpu.CompilerParams(use_tc_tiling_on_sc=False, needs_layout_passes=False),
        out_type=(
            jax.ShapeDtypeStruct((E_PAD, 16), jnp.float32),
            jax.ShapeDtypeStruct((E_PAD, 16), jnp.float32),
        ),
        scratch_types=[
            pltpu.VMEM((GCH,), jnp.int32),
            pltpu.VMEM((GCH, 16), jnp.float32),
            pltpu.SemaphoreType.DMA,
        ],
    )
    def run(pos_hbm, send_hbm, recv_hbm, out_s, out_r, idx_v, rows_v, sem):
        wid = lax.axis_index("s") * NC + lax.axis_index("c")
        base = wid * EPW

        def body(g, _):
            off = base + g * GCH
            pltpu.sync_copy(send_hbm.at[pl.ds(off, GCH)], idx_v)
            pltpu.async_copy(pos_hbm.at[idx_v], rows_v, sem).wait()
            pltpu.sync_copy(rows_v, out_s.at[pl.ds(off, GCH)])
            pltpu.sync_copy(recv_hbm.at[pl.ds(off, GCH)], idx_v)
            pltpu.async_copy(pos_hbm.at[idx_v], rows_v, sem).wait()
            pltpu.sync_copy(rows_v, out_r.at[pl.ds(off, GCH)])
            return _

        lax.fori_loop(0, EPW // GCH, body, 0)

    return run(pos_pad, send_p, recv_p)


def _silu(x):
    return x / (1.0 + jnp.exp(-x))


def _edge_w_kernel(ps_ref, pr_ref, m1_ref, m2_ref, w1_ref, w2_ref):
    vec = pr_ref[...] - ps_ref[...]
    vt = jnp.transpose(vec)
    x = vt[0:1, :]
    y = vt[1:2, :]
    z = vt[2:3, :]
    r = jnp.sqrt(x * x + y * y + z * z) + 1e-9
    xu = x / r
    yu = y / r
    zu = z / r

    s3 = 3.0 ** 0.5
    s15 = 15.0 ** 0.5
    one = jnp.ones_like(xu)
    sh_rows = [
        one,
        s3 * xu, s3 * yu, s3 * zu,
        s15 * xu * yu, s15 * yu * zu,
        (5.0 ** 0.5 / 2.0) * (3.0 * zu * zu - 1.0),
        s15 * xu * zu, (s15 / 2.0) * (xu * xu - yu * yu),
        (35.0 / 8.0) ** 0.5 * yu * (3.0 * xu * xu - yu * yu),
        (105.0 ** 0.5) * xu * yu * zu,
        (21.0 / 8.0) ** 0.5 * yu * (5.0 * zu * zu - 1.0),
        (7.0 ** 0.5 / 2.0) * (5.0 * zu * zu * zu - 3.0 * zu),
        (21.0 / 8.0) ** 0.5 * xu * (5.0 * zu * zu - 1.0),
        (105.0 ** 0.5 / 2.0) * zu * (xu * xu - yu * yu),
        (35.0 / 8.0) ** 0.5 * xu * (xu * xu - 3.0 * yu * yu),
    ]
    sh = jnp.concatenate(sh_rows, axis=0)

    scale = (2.0 / R_MAX) ** 0.5
    n8 = (lax.broadcasted_iota(jnp.int32, (NUM_BESSEL, 1), 0) + 1).astype(
        jnp.float32)
    rb = jnp.broadcast_to(r, (NUM_BESSEL, r.shape[1]))
    bess = jnp.sin(rb * (n8 * (jnp.pi / R_MAX))) * (scale / rb)
    u = r / R_MAX
    p = P_CUT
    u2 = u * u
    u4 = u2 * u2
    u5 = u4 * u
    u6 = u4 * u2
    u7 = u6 * u
    env = (1.0
           - (p + 1.0) * (p + 2.0) / 2.0 * u5
           + p * (p + 2.0) * u6
           - p * (p + 1.0) / 2.0 * u7)
    env = jnp.where(u < 1.0, env, 0.0)
    ef = bess * env

    for m_ref, w_ref in ((m1_ref, w1_ref), (m2_ref, w2_ref)):
        m = m_ref[...]
        a1t = m[:, 0:8]
        a2t = m[:, 8:72]
        a3t = m[:, 72:136]
        a4t = m[0:16, 136:200]
        h = _silu(jnp.dot(a1t, ef, preferred_element_type=jnp.float32))
        h = _silu(jnp.dot(a2t, h, preferred_element_type=jnp.float32))
        h = _silu(jnp.dot(a3t, h, preferred_element_type=jnp.float32))
        rr = jnp.dot(a4t, h, preferred_element_type=jnp.float32)
        w_ref[...] = jnp.transpose(sh * rr)


def _edge_w(pos_s, pos_r, mlp1, mlp2):
    be = 1024
    grid = (E_PAD // be,)
    return pl.pallas_call(
        _edge_w_kernel,
        grid=grid,
        in_specs=[
            pl.BlockSpec((be, 16), lambda i: (i, 0)),
            pl.BlockSpec((be, 16), lambda i: (i, 0)),
            pl.BlockSpec((64, 200), lambda i: (0, 0)),
            pl.BlockSpec((64, 200), lambda i: (0, 0)),
        ],
        out_specs=[
            pl.BlockSpec((be, K), lambda i: (i, 0)),
            pl.BlockSpec((be, K), lambda i: (i, 0)),
        ],
        out_shape=[
            jax.ShapeDtypeStruct((E_PAD, K), jnp.float32),
            jax.ShapeDtypeStruct((E_PAD, K), jnp.float32),
        ],
    )(pos_s, pos_r, mlp1, mlp2)


def _embed_kernel(sp_ref, w_ref, o_ref):
    sp = sp_ref[...]
    io = lax.broadcasted_iota(jnp.int32, (sp.shape[0], 16), 1).astype(
        jnp.float32)
    oh = (sp == io).astype(jnp.float32)
    o_ref[...] = jnp.dot(oh, w_ref[...], preferred_element_type=jnp.float32)


def _embed(species_f, w_pad):
    bn = 1024
    return pl.pallas_call(
        _embed_kernel,
        grid=(N_PAD // bn,),
        in_specs=[
            pl.BlockSpec((bn, 1), lambda i: (i, 0)),
            pl.BlockSpec((16, C), lambda i: (0, 0)),
        ],
        out_specs=pl.BlockSpec((bn, C), lambda i: (i, 0)),
        out_shape=jax.ShapeDtypeStruct((N_PAD, C), jnp.float32),
    )(species_f, w_pad)


def _msg_agg(feats, send_p, recv_p, w_pad):
    mesh = plsc.VectorSubcoreMesh(core_axis_name="c", subcore_axis_name="s")
    mbuf = EPT + BS
    rpt = CH // NS

    @functools.partial(
        pl.kernel,
        mesh=mesh,
        compiler_params=pltpu.CompilerParams(
            use_tc_tiling_on_sc=False, needs_layout_passes=False),
        out_type=jax.ShapeDtypeStruct((N_PAD, K * C), jnp.float32),
        scratch_types=[
            pltpu.VMEM((mbuf,), jnp.int32),
            pltpu.VMEM((mbuf,), jnp.int32),
            pltpu.VMEM((mbuf,), jnp.int32),
            pltpu.VMEM((BS, C), jnp.float32),
            pltpu.VMEM((BS, K), jnp.float32),
            pltpu.VMEM((BS, K * C), jnp.float32),
            pltpu.VMEM((2, K * C), jnp.float32),
            pltpu.VMEM_SHARED((CH, K * C), jnp.float32),
            pltpu.SemaphoreType.DMA,
            pltpu.SemaphoreType.DMA,
        ],
    )
    def run(feats_hbm, send_hbm, recv_hbm, w_hbm, agg_hbm,
            sendbuf, recvbuf, meid, srcb, wb, msg, zbuf,
            agg_s, sem1, sem2):
        cc = lax.axis_index("c")
        ss = lax.axis_index("s")
        tilebase = ss * EPT

        def zinit(j, _):
            for rr in range(2):
                zbuf[rr, pl.ds(j * LANES, LANES)] = jnp.zeros(
                    (LANES,), jnp.float32)
            return _
        lax.fori_loop(0, (K * C) // LANES, zinit, 0)

        pltpu.sync_copy(send_hbm.at[pl.ds(tilebase, EPT)],
                        sendbuf.at[pl.ds(0, EPT)])
        pltpu.sync_copy(recv_hbm.at[pl.ds(tilebase, EPT)],
                        recvbuf.at[pl.ds(0, EPT)])
        sendbuf[pl.ds(EPT, LANES)] = jnp.zeros((LANES,), jnp.int32)

        def pass_body(p_idx, _):
            chunk = NC * p_idx + cc
            nbase = chunk * CH
            recvbuf[pl.ds(EPT, LANES)] = jnp.full((LANES,), nbase, jnp.int32)

            def zrow(i, _):
                pltpu.sync_copy(zbuf, agg_s.at[pl.ds(ss * rpt + i * 2, 2)])
                return _
            lax.fori_loop(0, rpt // 2, zrow, 0)
            plsc.subcore_barrier()

            def scan_body(g, cnt):
                rv = recvbuf[pl.ds(g * LANES, LANES)]
                m = (rv >= nbase) & (rv < nbase + CH)
                mi = m.astype(jnp.int32)
                dest = cnt + plsc.cumsum(mi) - 1
                eid = (tilebase + g * LANES) + lax.iota(jnp.int32, LANES)
                plsc.store_scatter(meid, [dest], eid, mask=m)
                return cnt + jnp.sum(mi)

            cnt = lax.fori_loop(0, EPT // LANES, scan_body, 0)

            meid[pl.ds(cnt, LANES)] = jnp.full((LANES,), E_PAD, jnp.int32)

            nb = (cnt + BS - 1) // BS

            def proc_body(b, _):
                off = b * BS
                eidx = meid[pl.ds(off, BS)]
                rel = jnp.minimum(eidx - tilebase, EPT)
                sidx = plsc.load_gather(sendbuf, [rel])
                ridx = plsc.load_gather(recvbuf, [rel]) - nbase
                cp1 = pltpu.async_copy(feats_hbm.at[sidx], srcb, sem1)
                cp2 = pltpu.async_copy(w_hbm.at[eidx], wb, sem2)
                cp1.wait()
                cp2.wait()

                def edge_body(e, _):
                    srow = [srcb[e, pl.ds(j * LANES, LANES)] for j in range(8)]
                    for k in range(K):
                        wspl = plsc.load_gather(
                            wb, [jnp.full((LANES,), e, jnp.int32),
                                 jnp.full((LANES,), k, jnp.int32)])
                        for j in range(8):
                            msg[e, pl.ds(k * C + j * LANES, LANES)] = (
                                wspl * srow[j])
                    return _

                lax.fori_loop(0, BS, edge_body, 0)
                pltpu.sync_copy(msg, agg_s.at[ridx], add=True)
                return _

            lax.fori_loop(0, nb, proc_body, 0)
            plsc.subcore_barrier()

            pltpu.sync_copy(
                agg_s.at[pl.ds(ss * rpt, rpt)],
                agg_hbm.at[pl.ds(nbase + ss * rpt, rpt)])
            return _

        lax.fori_loop(0, NCHUNK // NC, pass_body, 0)

    return run(feats, send_p, recv_p, w_pad)


def _node_update_kernel(agg_ref, f_ref, pp_ref, l_ref, s_ref, o_ref):
    inv_avg = 1.0 / AVG_N
    a = agg_ref[...] * inv_avg
    pp = pp_ref[...]
    p2 = pp[0:1, :]
    p3 = pp[1:2, :]

    m0 = a[:, 0:C]
    inv0 = m0 * m0
    inv1 = (a[:, C:2 * C] * a[:, C:2 * C]
            + a[:, 2 * C:3 * C] * a[:, 2 * C:3 * C]
            + a[:, 3 * C:4 * C] * a[:, 3 * C:4 * C])
    inv2 = sum(a[:, k * C:(k + 1) * C] * a[:, k * C:(k + 1) * C]
               for k in range(4, 9))
    inv3 = sum(a[:, k * C:(k + 1) * C] * a[:, k * C:(k + 1) * C]
               for k in range(9, 16))
    b = (m0 + inv0 * pp[2:3, :] + inv1 * pp[3:4, :]
         + inv2 * pp[4:5, :] + inv3 * pp[5:6, :])
    b = b + (b * b) * p2 + (b * b * b) * p3
    bl = jnp.dot(b, l_ref[...], preferred_element_type=jnp.float32)
    f = f_ref[...]
    o_ref[...] = _silu(bl) + jnp.dot(f, s_ref[...],
                                     preferred_element_type=jnp.float32)


def _node_update(agg, feats, pp, l_mat, s_mat):
    bn = 256
    return pl.pallas_call(
        _node_update_kernel,
        grid=(N_PAD // bn,),
        in_specs=[
            pl.BlockSpec((bn, K * C), lambda i: (i, 0)),
            pl.BlockSpec((bn, C), lambda i: (i, 0)),
            pl.BlockSpec((8, C), lambda i: (0, 0)),
            pl.BlockSpec((C, C), lambda i: (0, 0)),
            pl.BlockSpec((C, C), lambda i: (0, 0)),
        ],
        out_specs=pl.BlockSpec((bn, C), lambda i: (i, 0)),
        out_shape=jax.ShapeDtypeStruct((N_PAD, C), jnp.float32),
    )(agg, feats, pp, l_mat, s_mat)


def _readout_kernel(h1_ref, h2_ref, sp_ref, ae_ref, r1_ref, r2a_ref, r2b_ref,
                    o_ref):
    sp = sp_ref[...]
    io = lax.broadcasted_iota(jnp.int32, (sp.shape[0], 16), 1).astype(
        jnp.float32)
    oh = (sp == io).astype(jnp.float32)
    e0 = jnp.sum(oh * ae_ref[...], axis=1, keepdims=True)
    e1 = jnp.sum(h1_ref[...] * r1_ref[...], axis=1, keepdims=True)
    t = _silu(jnp.dot(h2_ref[...], r2a_ref[...],
                      preferred_element_type=jnp.float32))
    e2 = jnp.sum(t * r2b_ref[...], axis=1, keepdims=True)
    o_ref[...] = e0 + e1 + e2


def _readout(h1, h2, species_f, ae_row, r1_row, r2a, r2b_row):
    bn = 1024
    return pl.pallas_call(
        _readout_kernel,
        grid=(N_PAD // bn,),
        in_specs=[
            pl.BlockSpec((bn, C), lambda i: (i, 0)),
            pl.BlockSpec((bn, C), lambda i: (i, 0)),
            pl.BlockSpec((bn, 1), lambda i: (i, 0)),
            pl.BlockSpec((1, 16), lambda i: (0, 0)),
            pl.BlockSpec((1, C), lambda i: (0, 0)),
            pl.BlockSpec((C, 16), lambda i: (0, 0)),
            pl.BlockSpec((1, 16), lambda i: (0, 0)),
        ],
        out_specs=pl.BlockSpec((bn, 1), lambda i: (i, 0)),
        out_shape=jax.ShapeDtypeStruct((N_PAD, 1), jnp.float32),
    )(h1, h2, species_f, ae_row, r1_row, r2a, r2b_row)


def _pack_mlp(p):
    m = jnp.zeros((64, 200), jnp.float32)
    m = m.at[:, 0:8].set(p['A1'].T)
    m = m.at[:, 8:72].set(p['A2'].T)
    m = m.at[:, 72:136].set(p['A3'].T)
    m = m.at[0:16, 136:200].set(p['A4'].T)
    return m


def _pack_pp(p):
    pp = jnp.zeros((8, C), jnp.float32)
    pp = pp.at[0, :].set(p['P2'])
    pp = pp.at[1, :].set(p['P3'])
    pp = pp.at[2:6, :].set(p['C'])
    return pp


def kernel(positions, species, edge_index, params):
    f32 = jnp.float32
    i32 = jnp.int32
    sender = edge_index[0].astype(i32)
    receiver = edge_index[1].astype(i32)
    epad = E_PAD - N_EDGES
    send_p = jnp.concatenate([sender, jnp.zeros((epad,), i32)])
    recv_p = jnp.concatenate(
        [receiver, jnp.full((epad,), RECV_SENTINEL, i32)])

    pos_pad = jnp.pad(positions.astype(f32),
                      ((0, N_PAD - N_NODES), (0, 13)))
    species_f = jnp.pad(species.astype(f32), (0, N_PAD - N_NODES))[:, None]

    p = params
    w_embed_pad = jnp.pad(p['W_embed'].astype(f32), ((0, 6), (0, 0)))
    ae_row = jnp.pad(p['AE'].astype(f32), (0, 6))[None, :]
    r1_row = p['R1'].astype(f32)[:, 0][None, :]
    r2b_row = jnp.pad(p['R2b'].astype(f32)[:, 0], (0, 0))[None, :]

    mlp1 = _pack_mlp(p['int1'])
    mlp2 = _pack_mlp(p['int2'])
    pp1 = _pack_pp(p['int1'])
    pp2 = _pack_pp(p['int2'])

    pos_s, pos_r = _pos_gather(pos_pad, send_p, recv_p)
    w1, w2 = _edge_w(pos_s, pos_r, mlp1, mlp2)
    w1p = jnp.pad(w1, ((0, BS), (0, 0)))
    w2p = jnp.pad(w2, ((0, BS), (0, 0)))

    feats0 = _embed(species_f, w_embed_pad)
    agg1 = _msg_agg(feats0, send_p, recv_p, w1p)
    h1 = _node_update(agg1, feats0, pp1, p['int1']['L'], p['int1']['S'])
    agg2 = _msg_agg(h1, send_p, recv_p, w2p)
    h2 = _node_update(agg2, h1, pp2, p['int2']['L'], p['int2']['S'])

    out = _readout(h1, h2, species_f, ae_row, r1_row, p['R2a'], r2b_row)
    return out[:N_NODES, 0]
